# speedup vs baseline: 48.6070x; 48.6070x over previous
import jax
import jax.numpy as jnp
from jax.experimental import pallas as pallas

APn = 256
UEn = 2048
HID = 128
NS = 10


def _mm(a, b):
    return jax.lax.dot_general(a, b, (((1,), (0,)), ((), ())),
                               preferred_element_type=jnp.float32)


def _mm_nt(a, b):
    return jax.lax.dot_general(a, b, (((1,), (1,)), ((), ())),
                               preferred_element_type=jnp.float32)


def _build_mask(adj, n_rows, n_cols):
    iota = jax.lax.broadcasted_iota(jnp.int32, (n_rows, n_cols), 1)
    m = jnp.zeros((n_rows, n_cols), dtype=jnp.float32)
    for k in range(NS):
        col = adj[:, k][:, None]
        m = jnp.maximum(m, (col == iota).astype(jnp.float32))
    return m


def _fused_kernel(p_ref, adj_ue_ref, adj_ap_ref,
                  w1_ref, w2_ref, w3_ref, w4_ref, w5_ref, out_ref):
    P = p_ref[...]
    adj_ap = adj_ap_ref[...]
    adj_ue = adj_ue_ref[...]

    m1u = _build_mask(adj_ap, APn, UEn)
    m3u = _build_mask(adj_ue, UEn, APn)
    d1 = jnp.sum(m1u, axis=1, keepdims=True)
    d3 = jnp.sum(m3u, axis=1, keepdims=True)
    m1n = m1u / (d1 + 1.0)
    m3n = m3u / (d3 + 1.0)

    m2u = jnp.minimum(_mm(m1u, m3u), 1.0)
    d2 = jnp.sum(m2u, axis=1, keepdims=True)
    m2n = m2u / (d2 + 1.0)
    m4u = jnp.minimum(_mm(m3u, m1u), 1.0)
    d4 = jnp.sum(m4u, axis=1, keepdims=True)
    m4n = m4u / (d4 + 1.0)

    W1 = w1_ref[...]; W2 = w2_ref[...]; W3 = w3_ref[...]
    W4 = w4_ref[...]; W5 = w5_ref[...]

    x1 = jax.nn.relu(_mm_nt(W1[:, :UEn], P) +
                     _mm_nt(_mm(W1[:, UEn:], P), m1n))
    x2 = jax.nn.relu(_mm_nt(W2[:, :UEn], P) +
                     _mm_nt(_mm_nt(W2[:, UEn:], P), m2n))
    x3 = jax.nn.relu(_mm(W3[:, :APn], P) +
                     _mm_nt(_mm_nt(W3[:, APn:], P), m3n))
    x4 = jax.nn.relu(_mm(W4[:, :APn], P) +
                     _mm_nt(_mm(W4[:, APn:], P), m4n))

    cat12 = jnp.concatenate([x1, x2], axis=0)
    cat34 = jnp.concatenate([x3, x4], axis=0)
    x5 = jax.nn.relu(_mm(W5[:, :2 * HID], cat12) +
                     _mm(W5[:, 2 * HID:], _mm_nt(cat34, m1n)))
    out_ref[...] = x5


def kernel(pl, require, adj_ue, adj_ap, W1, W2, W3, W4, W5):
    del require
    return pallas.pallas_call(
        _fused_kernel,
        out_shape=jax.ShapeDtypeStruct((HID, APn), jnp.float32),
    )(pl, adj_ue, adj_ap, W1, W2, W3, W4, W5)
